# Initial kernel scaffold; baseline (speedup 1.0000x reference)
import jax
import jax.numpy as jnp
from jax.experimental import pallas as pl


def kernel(x1, edge_index1, batch1, x2, edge_index2, batch2, W1, b1, W2, b2, W3, b3, fc1_W, fc1_b, fc2_W, fc2_b, fc3_W, fc3_b):
    raise NotImplementedError("write your pallas kernel here")


# speedup vs baseline: 7.9742x; 7.9742x over previous
import functools

import jax
import jax.numpy as jnp
from jax import lax
from jax.experimental import pallas as pl
from jax.experimental.pallas import tpu as pltpu
from jax.experimental.pallas import tpu_sc as plsc

G = 64


_NSUB = 16
_CHUNK = 128


def _sc_mesh():
    return plsc.VectorSubcoreMesh(core_axis_name="c", subcore_axis_name="s")


def _make_edge_kernel(C, A, Dd):
    nz = A // (_NSUB * _CHUNK)

    @functools.partial(
        pl.kernel,
        mesh=_sc_mesh(),
        out_type=jax.ShapeDtypeStruct((2, A, Dd), jnp.float32),
        scratch_types=[
            pltpu.VMEM((_CHUNK,), jnp.int32),
            pltpu.VMEM((_CHUNK,), jnp.int32),
            pltpu.VMEM((_CHUNK, Dd), jnp.float32),
            pltpu.VMEM_SHARED((A, Dd), jnp.float32),
            pltpu.SemaphoreType.DMA,
        ],
    )
    def edge_kernel(y_hbm, src_hbm, dst_hbm, zeros_hbm, acc_out, src_v, dst_v,
                    rows_v, acc, sem):
        c = lax.axis_index("c")
        s = lax.axis_index("s")
        pltpu.sync_copy(zeros_hbm, rows_v)
        for k in range(nz):
            pltpu.sync_copy(rows_v, acc.at[pl.ds((s * nz + k) * _CHUNK, _CHUNK)])
        plsc.subcore_barrier()

        def body(j, carry):
            pltpu.sync_copy(src_hbm.at[c, s, j], src_v)
            pltpu.sync_copy(dst_hbm.at[c, s, j], dst_v)
            pltpu.async_copy(y_hbm.at[src_v], rows_v, sem).wait()
            pltpu.sync_copy(rows_v, acc.at[dst_v], add=True)
            return carry

        lax.fori_loop(0, C, body, 0)
        plsc.subcore_barrier()
        for k in range(nz):
            r0 = (s * nz + k) * _CHUNK
            pltpu.sync_copy(acc.at[pl.ds(r0, _CHUNK)], rows_v)
            pltpu.sync_copy(rows_v, acc_out.at[c, pl.ds(r0, _CHUNK)])

    return edge_kernel


_BR = 1024


def _dinv(d_ref):
    return lax.rsqrt(jnp.maximum(d_ref[:, 0:1], 1.0))


def _mm_pre_body(x_ref, d_ref, w_ref, o_ref):
    o_ref[...] = jnp.dot(x_ref[...] * _dinv(d_ref), w_ref[...],
                         preferred_element_type=jnp.float32)


def _mm_mid_body(a_ref, d_ref, b_ref, w_ref, o_ref):
    dinv = _dinv(d_ref)
    h = jnp.maximum(a_ref[...] * dinv + b_ref[...], 0.0)
    o_ref[...] = jnp.dot(h * dinv, w_ref[...],
                         preferred_element_type=jnp.float32)


def _pool_body(a_ref, d_ref, b_ref, batch_ref, o_ref, acc_ref):
    i = pl.program_id(0)

    @pl.when(i == 0)
    def _():
        acc_ref[...] = jnp.zeros_like(acc_ref)

    h3 = a_ref[...] * _dinv(d_ref) + b_ref[...]
    he = jnp.concatenate(
        [h3, jnp.ones((h3.shape[0], 128), jnp.float32)], axis=1)
    seg = batch_ref[0]
    qt = (lax.broadcasted_iota(jnp.int32, (128, h3.shape[0]), 0)
          == seg).astype(jnp.float32)
    acc_ref[...] += jnp.dot(qt, he, preferred_element_type=jnp.float32)

    @pl.when(i == pl.num_programs(0) - 1)
    def _():
        o_ref[...] = acc_ref[...]


def _head_body(p_ref, w1_ref, b1_ref, w2_ref, b2_ref, w3_ref, b3_ref, o_ref):
    pooled = p_ref[...]
    cnt = jnp.maximum(pooled[:, 128:129], 1.0)
    mean = pooled[:, :128] / cnt
    h = jnp.concatenate([mean[:G], mean[G:2 * G]], axis=1)
    h = jnp.maximum(jnp.dot(h, w1_ref[...],
                            preferred_element_type=jnp.float32) + b1_ref[...],
                    0.0)
    h = jnp.maximum(jnp.dot(h, w2_ref[...],
                            preferred_element_type=jnp.float32) + b2_ref[...],
                    0.0)
    o_ref[...] = jnp.dot(h, w3_ref[...],
                         preferred_element_type=jnp.float32) + b3_ref[...]


def kernel(x1, edge_index1, batch1, x2, edge_index2, batch2,
           W1, b1, W2, b2, W3, b3,
           fc1_W, fc1_b, fc2_W, fc2_b, fc3_W, fc3_b):
    N, D = x1.shape
    H = W1.shape[1]
    E = edge_index1.shape[1]
    Etot = E + N
    C = -(-Etot // (_NSUB * _CHUNK))
    EP = _NSUB * C * _CHUNK
    A = (N // (_NSUB * _CHUNK) + 1) * _NSUB * _CHUNK

    loop = jnp.arange(N, dtype=jnp.int32)

    def prep(ei, off):
        src = jnp.concatenate([ei[0] + off, loop + off])
        dst = jnp.concatenate([ei[1], loop])
        src = jnp.pad(src, (0, EP - Etot), constant_values=off)
        dst = jnp.pad(dst, (0, EP - Etot), constant_values=N)
        return src.reshape(_NSUB, C, _CHUNK), dst.reshape(_NSUB, C, _CHUNK)

    s1, d1 = prep(edge_index1, 0)
    s2, d2 = prep(edge_index2, A)
    src_all = jnp.stack([s1, s2])
    dst_all = jnp.stack([d1, d2])

    zerosD = jnp.zeros((_CHUNK, D), jnp.float32)
    edge_kernel = _make_edge_kernel(C, A, D)

    ones_tab = jnp.ones((2 * A, D), jnp.float32)
    deg16 = edge_kernel(ones_tab, src_all, dst_all, zerosD).reshape(2 * A, D)

    x_c = jnp.zeros((2, A, D), jnp.float32)
    x_c = x_c.at[0, :N].set(x1).at[1, :N].set(x2).reshape(2 * A, D)
    nblk = (2 * A) // _BR

    def run_pre(x, W):
        return pl.pallas_call(
            _mm_pre_body,
            grid=(nblk,),
            in_specs=[
                pl.BlockSpec((_BR, D), lambda i: (i, 0)),
                pl.BlockSpec((_BR, D), lambda i: (i, 0)),
                pl.BlockSpec((D, H), lambda i: (0, 0)),
            ],
            out_specs=pl.BlockSpec((_BR, H), lambda i: (i, 0)),
            out_shape=jax.ShapeDtypeStruct((2 * A, H), jnp.float32),
        )(x, deg16, W)

    def run_mid(a, b, W):
        return pl.pallas_call(
            _mm_mid_body,
            grid=(nblk,),
            in_specs=[
                pl.BlockSpec((_BR, H), lambda i: (i, 0)),
                pl.BlockSpec((_BR, H), lambda i: (i, 0)),
                pl.BlockSpec((1, H), lambda i: (0, 0)),
                pl.BlockSpec((H, H), lambda i: (0, 0)),
            ],
            out_specs=pl.BlockSpec((_BR, H), lambda i: (i, 0)),
            out_shape=jax.ShapeDtypeStruct((2 * A, H), jnp.float32),
        )(a, deg16, b.reshape(1, H), W)

    y1 = run_pre(x_c, W1)
    a1 = edge_kernel(y1, src_all, dst_all, zerosD).reshape(2 * A, H)
    y2 = run_mid(a1, b1, W2)
    a2 = edge_kernel(y2, src_all, dst_all, zerosD).reshape(2 * A, H)
    y3 = run_mid(a2, b2, W3)
    a3 = edge_kernel(y3, src_all, dst_all, zerosD).reshape(2 * A, H)

    batch_c = jnp.full((2, A), 1000, jnp.int32)
    batch_c = (batch_c.at[0, :N].set(batch1).at[1, :N].set(batch2 + G)
               .reshape(nblk, 1, _BR))
    pooled = pl.pallas_call(
        _pool_body,
        grid=(nblk,),
        in_specs=[
            pl.BlockSpec((_BR, H), lambda i: (i, 0)),
            pl.BlockSpec((_BR, H), lambda i: (i, 0)),
            pl.BlockSpec((1, H), lambda i: (0, 0)),
            pl.BlockSpec((1, 1, _BR), lambda i: (i, 0, 0)),
        ],
        out_specs=pl.BlockSpec((128, H + 128), lambda i: (0, 0)),
        out_shape=jax.ShapeDtypeStruct((128, H + 128), jnp.float32),
        scratch_shapes=[pltpu.VMEM((128, H + 128), jnp.float32)],
    )(a3, deg16, b3.reshape(1, H), batch_c)

    OUT = fc3_W.shape[1]
    fc3_Wp = jnp.pad(fc3_W, ((0, 0), (0, 128 - OUT)))
    fc3_bp = jnp.pad(fc3_b, (0, 128 - OUT)).reshape(1, 128)
    out = pl.pallas_call(
        _head_body,
        out_shape=jax.ShapeDtypeStruct((G, 128), jnp.float32),
    )(pooled, fc1_W, fc1_b.reshape(1, -1), fc2_W, fc2_b.reshape(1, -1),
      fc3_Wp, fc3_bp)
    return out[:, :OUT]


# speedup vs baseline: 10.6186x; 1.3316x over previous
import functools

import jax
import jax.numpy as jnp
from jax import lax
from jax.experimental import pallas as pl
from jax.experimental.pallas import tpu as pltpu
from jax.experimental.pallas import tpu_sc as plsc

G = 64


_NSUB = 16
_CHUNK = 128


def _sc_mesh():
    return plsc.VectorSubcoreMesh(core_axis_name="c", subcore_axis_name="s")


def _make_edge_kernel(C, A, Dd):
    nz = A // (_NSUB * _CHUNK)

    @functools.partial(
        pl.kernel,
        mesh=_sc_mesh(),
        out_type=jax.ShapeDtypeStruct((2, A, Dd), jnp.float32),
        scratch_types=[
            pltpu.VMEM((_CHUNK,), jnp.int32),
            pltpu.VMEM((_CHUNK,), jnp.int32),
            pltpu.VMEM((_CHUNK,), jnp.int32),
            pltpu.VMEM((_CHUNK,), jnp.int32),
            pltpu.VMEM((_CHUNK, Dd), jnp.float32),
            pltpu.VMEM((_CHUNK, Dd), jnp.float32),
            pltpu.VMEM_SHARED((A, Dd), jnp.float32),
            pltpu.SemaphoreType.DMA,
            pltpu.SemaphoreType.DMA,
            pltpu.SemaphoreType.DMA,
            pltpu.SemaphoreType.DMA,
            pltpu.SemaphoreType.DMA,
            pltpu.SemaphoreType.DMA,
        ],
    )
    def edge_kernel(y_hbm, src_hbm, dst_hbm, zeros_hbm, acc_out,
                    srcA, dstA, srcB, dstB, rowsA, rowsB, acc,
                    gsA, gsB, sSA, sDA, sSB, sDB):
        c = lax.axis_index("c")
        s = lax.axis_index("s")
        pltpu.sync_copy(zeros_hbm, rowsA)
        for k in range(nz):
            pltpu.sync_copy(rowsA, acc.at[pl.ds((s * nz + k) * _CHUNK, _CHUNK)])
        pltpu.async_copy(src_hbm.at[c, s, 0], srcA, sSA)
        pltpu.async_copy(dst_hbm.at[c, s, 0], dstA, sDA)
        pltpu.async_copy(src_hbm.at[c, s, 1], srcB, sSB)
        pltpu.async_copy(dst_hbm.at[c, s, 1], dstB, sDB)
        plsc.subcore_barrier()
        pltpu.make_async_copy(src_hbm.at[c, s, 0], srcA, sSA).wait()
        pltpu.make_async_copy(dst_hbm.at[c, s, 0], dstA, sDA).wait()
        pltpu.async_copy(y_hbm.at[srcA], rowsA, gsA)

        def body(t, carry):
            j0 = 2 * t
            pltpu.make_async_copy(y_hbm.at[srcA], rowsA, gsA).wait()
            pltpu.make_async_copy(src_hbm.at[c, s, 0], srcB, sSB).wait()
            pltpu.make_async_copy(dst_hbm.at[c, s, 0], dstB, sDB).wait()
            pltpu.async_copy(y_hbm.at[srcB], rowsB, gsB)
            pltpu.sync_copy(rowsA, acc.at[dstA], add=True)
            pltpu.async_copy(src_hbm.at[c, s, j0 + 2], srcA, sSA)
            pltpu.async_copy(dst_hbm.at[c, s, j0 + 2], dstA, sDA)
            pltpu.make_async_copy(y_hbm.at[srcB], rowsB, gsB).wait()
            pltpu.make_async_copy(src_hbm.at[c, s, 0], srcA, sSA).wait()
            pltpu.make_async_copy(dst_hbm.at[c, s, 0], dstA, sDA).wait()
            pltpu.async_copy(y_hbm.at[srcA], rowsA, gsA)
            pltpu.sync_copy(rowsB, acc.at[dstB], add=True)
            pltpu.async_copy(src_hbm.at[c, s, j0 + 3], srcB, sSB)
            pltpu.async_copy(dst_hbm.at[c, s, j0 + 3], dstB, sDB)
            return carry

        lax.fori_loop(0, C // 2, body, 0)
        pltpu.make_async_copy(y_hbm.at[srcA], rowsA, gsA).wait()
        pltpu.make_async_copy(src_hbm.at[c, s, 0], srcB, sSB).wait()
        pltpu.make_async_copy(dst_hbm.at[c, s, 0], dstB, sDB).wait()
        plsc.subcore_barrier()
        for k in range(nz):
            r0 = (s * nz + k) * _CHUNK
            pltpu.sync_copy(acc.at[pl.ds(r0, _CHUNK)], rowsA)
            pltpu.sync_copy(rowsA, acc_out.at[c, pl.ds(r0, _CHUNK)])

    return edge_kernel


_BR = 1024


def _dinv(d_ref):
    return lax.rsqrt(jnp.maximum(d_ref[:, 0:1], 1.0))


def _mm_pre_body(x_ref, d_ref, w_ref, o_ref):
    o_ref[...] = jnp.dot(x_ref[...] * _dinv(d_ref), w_ref[...],
                         preferred_element_type=jnp.float32)


def _mm_mid_body(a_ref, d_ref, b_ref, w_ref, o_ref):
    dinv = _dinv(d_ref)
    h = jnp.maximum(a_ref[...] * dinv + b_ref[...], 0.0)
    o_ref[...] = jnp.dot(h * dinv, w_ref[...],
                         preferred_element_type=jnp.float32)


def _pool_body(a_ref, d_ref, b_ref, batch_ref, o_ref, acc_ref):
    i = pl.program_id(0)

    @pl.when(i == 0)
    def _():
        acc_ref[...] = jnp.zeros_like(acc_ref)

    h3 = a_ref[...] * _dinv(d_ref) + b_ref[...]
    he = jnp.concatenate(
        [h3, jnp.ones((h3.shape[0], 128), jnp.float32)], axis=1)
    seg = batch_ref[0]
    qt = (lax.broadcasted_iota(jnp.int32, (128, h3.shape[0]), 0)
          == seg).astype(jnp.float32)
    acc_ref[...] += jnp.dot(qt, he, preferred_element_type=jnp.float32)

    @pl.when(i == pl.num_programs(0) - 1)
    def _():
        o_ref[...] = acc_ref[...]


def _head_body(p_ref, w1_ref, b1_ref, w2_ref, b2_ref, w3_ref, b3_ref, o_ref):
    pooled = p_ref[...]
    cnt = jnp.maximum(pooled[:, 128:129], 1.0)
    mean = pooled[:, :128] / cnt
    h = jnp.concatenate([mean[:G], mean[G:2 * G]], axis=1)
    h = jnp.maximum(jnp.dot(h, w1_ref[...],
                            preferred_element_type=jnp.float32) + b1_ref[...],
                    0.0)
    h = jnp.maximum(jnp.dot(h, w2_ref[...],
                            preferred_element_type=jnp.float32) + b2_ref[...],
                    0.0)
    o_ref[...] = jnp.dot(h, w3_ref[...],
                         preferred_element_type=jnp.float32) + b3_ref[...]


def kernel(x1, edge_index1, batch1, x2, edge_index2, batch2,
           W1, b1, W2, b2, W3, b3,
           fc1_W, fc1_b, fc2_W, fc2_b, fc3_W, fc3_b):
    N, D = x1.shape
    H = W1.shape[1]
    E = edge_index1.shape[1]
    Etot = E + N
    C = -(-Etot // (_NSUB * _CHUNK))
    C += C % 2
    EP = _NSUB * C * _CHUNK
    A = (N // (_NSUB * _CHUNK) + 1) * _NSUB * _CHUNK

    loop = jnp.arange(N, dtype=jnp.int32)

    def prep(ei, off):
        src = jnp.concatenate([ei[0] + off, loop + off])
        dst = jnp.concatenate([ei[1], loop])
        src = jnp.pad(src, (0, EP - Etot), constant_values=off)
        dst = jnp.pad(dst, (0, EP - Etot), constant_values=N)
        src = jnp.concatenate(
            [src.reshape(_NSUB, C, _CHUNK),
             jnp.full((_NSUB, 2, _CHUNK), off, jnp.int32)], axis=1)
        dst = jnp.concatenate(
            [dst.reshape(_NSUB, C, _CHUNK),
             jnp.full((_NSUB, 2, _CHUNK), N, jnp.int32)], axis=1)
        return src, dst

    s1, d1 = prep(edge_index1, 0)
    s2, d2 = prep(edge_index2, A)
    src_all = jnp.stack([s1, s2])
    dst_all = jnp.stack([d1, d2])

    zerosD = jnp.zeros((_CHUNK, D), jnp.float32)
    edge_kernel = _make_edge_kernel(C, A, D)

    ones_tab = jnp.ones((2 * A, D), jnp.float32)
    deg16 = edge_kernel(ones_tab, src_all, dst_all, zerosD).reshape(2 * A, D)

    x_c = jnp.zeros((2, A, D), jnp.float32)
    x_c = x_c.at[0, :N].set(x1).at[1, :N].set(x2).reshape(2 * A, D)
    nblk = (2 * A) // _BR

    def run_pre(x, W):
        return pl.pallas_call(
            _mm_pre_body,
            grid=(nblk,),
            in_specs=[
                pl.BlockSpec((_BR, D), lambda i: (i, 0)),
                pl.BlockSpec((_BR, D), lambda i: (i, 0)),
                pl.BlockSpec((D, H), lambda i: (0, 0)),
            ],
            out_specs=pl.BlockSpec((_BR, H), lambda i: (i, 0)),
            out_shape=jax.ShapeDtypeStruct((2 * A, H), jnp.float32),
        )(x, deg16, W)

    def run_mid(a, b, W):
        return pl.pallas_call(
            _mm_mid_body,
            grid=(nblk,),
            in_specs=[
                pl.BlockSpec((_BR, H), lambda i: (i, 0)),
                pl.BlockSpec((_BR, H), lambda i: (i, 0)),
                pl.BlockSpec((1, H), lambda i: (0, 0)),
                pl.BlockSpec((H, H), lambda i: (0, 0)),
            ],
            out_specs=pl.BlockSpec((_BR, H), lambda i: (i, 0)),
            out_shape=jax.ShapeDtypeStruct((2 * A, H), jnp.float32),
        )(a, deg16, b.reshape(1, H), W)

    y1 = run_pre(x_c, W1)
    a1 = edge_kernel(y1, src_all, dst_all, zerosD).reshape(2 * A, H)
    y2 = run_mid(a1, b1, W2)
    a2 = edge_kernel(y2, src_all, dst_all, zerosD).reshape(2 * A, H)
    y3 = run_mid(a2, b2, W3)
    a3 = edge_kernel(y3, src_all, dst_all, zerosD).reshape(2 * A, H)

    batch_c = jnp.full((2, A), 1000, jnp.int32)
    batch_c = (batch_c.at[0, :N].set(batch1).at[1, :N].set(batch2 + G)
               .reshape(nblk, 1, _BR))
    pooled = pl.pallas_call(
        _pool_body,
        grid=(nblk,),
        in_specs=[
            pl.BlockSpec((_BR, H), lambda i: (i, 0)),
            pl.BlockSpec((_BR, H), lambda i: (i, 0)),
            pl.BlockSpec((1, H), lambda i: (0, 0)),
            pl.BlockSpec((1, 1, _BR), lambda i: (i, 0, 0)),
        ],
        out_specs=pl.BlockSpec((128, H + 128), lambda i: (0, 0)),
        out_shape=jax.ShapeDtypeStruct((128, H + 128), jnp.float32),
        scratch_shapes=[pltpu.VMEM((128, H + 128), jnp.float32)],
    )(a3, deg16, b3.reshape(1, H), batch_c)

    OUT = fc3_W.shape[1]
    fc3_Wp = jnp.pad(fc3_W, ((0, 0), (0, 128 - OUT)))
    fc3_bp = jnp.pad(fc3_b, (0, 128 - OUT)).reshape(1, 128)
    out = pl.pallas_call(
        _head_body,
        out_shape=jax.ShapeDtypeStruct((G, 128), jnp.float32),
    )(pooled, fc1_W, fc1_b.reshape(1, -1), fc2_W, fc2_b.reshape(1, -1),
      fc3_Wp, fc3_bp)
    return out[:, :OUT]


# speedup vs baseline: 10.8975x; 1.0263x over previous
import functools

import jax
import jax.numpy as jnp
from jax import lax
from jax.experimental import pallas as pl
from jax.experimental.pallas import tpu as pltpu
from jax.experimental.pallas import tpu_sc as plsc

G = 64


_NSUB = 16
_CHUNK = 128


def _sc_mesh():
    return plsc.VectorSubcoreMesh(core_axis_name="c", subcore_axis_name="s")


def _make_edge_kernel(C, A, Dd):
    nz = A // (_NSUB * _CHUNK)

    @functools.partial(
        pl.kernel,
        mesh=_sc_mesh(),
        out_type=jax.ShapeDtypeStruct((2, A, Dd), jnp.float32),
        scratch_types=[
            pltpu.VMEM((_CHUNK,), jnp.int32),
            pltpu.VMEM((_CHUNK,), jnp.int32),
            pltpu.VMEM((_CHUNK,), jnp.int32),
            pltpu.VMEM((_CHUNK,), jnp.int32),
            pltpu.VMEM((_CHUNK, Dd), jnp.float32),
            pltpu.VMEM((_CHUNK, Dd), jnp.float32),
            pltpu.VMEM_SHARED((A, Dd), jnp.float32),
            pltpu.SemaphoreType.DMA,
            pltpu.SemaphoreType.DMA,
            pltpu.SemaphoreType.DMA,
            pltpu.SemaphoreType.DMA,
            pltpu.SemaphoreType.DMA,
            pltpu.SemaphoreType.DMA,
        ],
    )
    def edge_kernel(y_hbm, src_hbm, dst_hbm, zeros_hbm, acc_out,
                    srcA, dstA, srcB, dstB, rowsA, rowsB, acc,
                    gsA, gsB, sSA, sDA, sSB, sDB):
        c = lax.axis_index("c")
        s = lax.axis_index("s")
        pltpu.sync_copy(zeros_hbm, rowsA)
        for k in range(nz):
            pltpu.sync_copy(rowsA, acc.at[pl.ds((s * nz + k) * _CHUNK, _CHUNK)])
        pltpu.async_copy(src_hbm.at[c, s, 0], srcA, sSA)
        pltpu.async_copy(dst_hbm.at[c, s, 0], dstA, sDA)
        pltpu.async_copy(src_hbm.at[c, s, 1], srcB, sSB)
        pltpu.async_copy(dst_hbm.at[c, s, 1], dstB, sDB)
        plsc.subcore_barrier()
        pltpu.make_async_copy(src_hbm.at[c, s, 0], srcA, sSA).wait()
        pltpu.make_async_copy(dst_hbm.at[c, s, 0], dstA, sDA).wait()
        pltpu.async_copy(y_hbm.at[srcA], rowsA, gsA)

        def body(t, carry):
            j0 = 2 * t
            pltpu.make_async_copy(src_hbm.at[c, s, 0], srcB, sSB).wait()
            pltpu.make_async_copy(dst_hbm.at[c, s, 0], dstB, sDB).wait()
            pltpu.async_copy(y_hbm.at[srcB], rowsB, gsB)
            pltpu.make_async_copy(y_hbm.at[srcA], rowsA, gsA).wait()
            pltpu.sync_copy(rowsA, acc.at[dstA], add=True)
            pltpu.async_copy(src_hbm.at[c, s, j0 + 2], srcA, sSA)
            pltpu.async_copy(dst_hbm.at[c, s, j0 + 2], dstA, sDA)
            pltpu.make_async_copy(src_hbm.at[c, s, 0], srcA, sSA).wait()
            pltpu.make_async_copy(dst_hbm.at[c, s, 0], dstA, sDA).wait()
            pltpu.async_copy(y_hbm.at[srcA], rowsA, gsA)
            pltpu.make_async_copy(y_hbm.at[srcB], rowsB, gsB).wait()
            pltpu.sync_copy(rowsB, acc.at[dstB], add=True)
            pltpu.async_copy(src_hbm.at[c, s, j0 + 3], srcB, sSB)
            pltpu.async_copy(dst_hbm.at[c, s, j0 + 3], dstB, sDB)
            return carry

        lax.fori_loop(0, C // 2, body, 0)
        pltpu.make_async_copy(y_hbm.at[srcA], rowsA, gsA).wait()
        pltpu.make_async_copy(src_hbm.at[c, s, 0], srcB, sSB).wait()
        pltpu.make_async_copy(dst_hbm.at[c, s, 0], dstB, sDB).wait()
        plsc.subcore_barrier()
        for k in range(nz):
            r0 = (s * nz + k) * _CHUNK
            pltpu.sync_copy(acc.at[pl.ds(r0, _CHUNK)], rowsA)
            pltpu.sync_copy(rowsA, acc_out.at[c, pl.ds(r0, _CHUNK)])

    return edge_kernel


_BR = 1024


def _dinv(d_ref):
    return lax.rsqrt(jnp.maximum(d_ref[:, 0:1], 1.0))


def _mm_pre_body(x_ref, d_ref, w_ref, o_ref):
    o_ref[...] = jnp.dot(x_ref[...] * _dinv(d_ref), w_ref[...],
                         preferred_element_type=jnp.float32)


def _mm_mid_body(a_ref, d_ref, b_ref, w_ref, o_ref):
    dinv = _dinv(d_ref)
    h = jnp.maximum(a_ref[...] * dinv + b_ref[...], 0.0)
    o_ref[...] = jnp.dot(h * dinv, w_ref[...],
                         preferred_element_type=jnp.float32)


def _pool_body(a_ref, d_ref, b_ref, batch_ref, o_ref, acc_ref):
    i = pl.program_id(0)

    @pl.when(i == 0)
    def _():
        acc_ref[...] = jnp.zeros_like(acc_ref)

    h3 = a_ref[...] * _dinv(d_ref) + b_ref[...]
    he = jnp.concatenate(
        [h3, jnp.ones((h3.shape[0], 128), jnp.float32)], axis=1)
    seg = batch_ref[0]
    qt = (lax.broadcasted_iota(jnp.int32, (128, h3.shape[0]), 0)
          == seg).astype(jnp.float32)
    acc_ref[...] += jnp.dot(qt, he, preferred_element_type=jnp.float32)

    @pl.when(i == pl.num_programs(0) - 1)
    def _():
        o_ref[...] = acc_ref[...]


def _head_body(p_ref, w1_ref, b1_ref, w2_ref, b2_ref, w3_ref, b3_ref, o_ref):
    pooled = p_ref[...]
    cnt = jnp.maximum(pooled[:, 128:129], 1.0)
    mean = pooled[:, :128] / cnt
    h = jnp.concatenate([mean[:G], mean[G:2 * G]], axis=1)
    h = jnp.maximum(jnp.dot(h, w1_ref[...],
                            preferred_element_type=jnp.float32) + b1_ref[...],
                    0.0)
    h = jnp.maximum(jnp.dot(h, w2_ref[...],
                            preferred_element_type=jnp.float32) + b2_ref[...],
                    0.0)
    o_ref[...] = jnp.dot(h, w3_ref[...],
                         preferred_element_type=jnp.float32) + b3_ref[...]


def kernel(x1, edge_index1, batch1, x2, edge_index2, batch2,
           W1, b1, W2, b2, W3, b3,
           fc1_W, fc1_b, fc2_W, fc2_b, fc3_W, fc3_b):
    N, D = x1.shape
    H = W1.shape[1]
    E = edge_index1.shape[1]
    Etot = E + N
    C = -(-Etot // (_NSUB * _CHUNK))
    C += C % 2
    EP = _NSUB * C * _CHUNK
    A = (N // (_NSUB * _CHUNK) + 1) * _NSUB * _CHUNK

    loop = jnp.arange(N, dtype=jnp.int32)

    def prep(ei, off):
        src = jnp.concatenate([ei[0] + off, loop + off])
        dst = jnp.concatenate([ei[1], loop])
        src = jnp.pad(src, (0, EP - Etot), constant_values=off)
        dst = jnp.pad(dst, (0, EP - Etot), constant_values=N)
        src = jnp.concatenate(
            [src.reshape(_NSUB, C, _CHUNK),
             jnp.full((_NSUB, 2, _CHUNK), off, jnp.int32)], axis=1)
        dst = jnp.concatenate(
            [dst.reshape(_NSUB, C, _CHUNK),
             jnp.full((_NSUB, 2, _CHUNK), N, jnp.int32)], axis=1)
        return src, dst

    s1, d1 = prep(edge_index1, 0)
    s2, d2 = prep(edge_index2, A)
    src_all = jnp.stack([s1, s2])
    dst_all = jnp.stack([d1, d2])

    zerosD = jnp.zeros((_CHUNK, D), jnp.float32)
    edge_kernel = _make_edge_kernel(C, A, D)

    ones_tab = jnp.ones((2 * A, D), jnp.float32)
    deg16 = edge_kernel(ones_tab, src_all, dst_all, zerosD).reshape(2 * A, D)

    x_c = jnp.zeros((2, A, D), jnp.float32)
    x_c = x_c.at[0, :N].set(x1).at[1, :N].set(x2).reshape(2 * A, D)
    nblk = (2 * A) // _BR

    def run_pre(x, W):
        return pl.pallas_call(
            _mm_pre_body,
            grid=(nblk,),
            in_specs=[
                pl.BlockSpec((_BR, D), lambda i: (i, 0)),
                pl.BlockSpec((_BR, D), lambda i: (i, 0)),
                pl.BlockSpec((D, H), lambda i: (0, 0)),
            ],
            out_specs=pl.BlockSpec((_BR, H), lambda i: (i, 0)),
            out_shape=jax.ShapeDtypeStruct((2 * A, H), jnp.float32),
        )(x, deg16, W)

    def run_mid(a, b, W):
        return pl.pallas_call(
            _mm_mid_body,
            grid=(nblk,),
            in_specs=[
                pl.BlockSpec((_BR, H), lambda i: (i, 0)),
                pl.BlockSpec((_BR, H), lambda i: (i, 0)),
                pl.BlockSpec((1, H), lambda i: (0, 0)),
                pl.BlockSpec((H, H), lambda i: (0, 0)),
            ],
            out_specs=pl.BlockSpec((_BR, H), lambda i: (i, 0)),
            out_shape=jax.ShapeDtypeStruct((2 * A, H), jnp.float32),
        )(a, deg16, b.reshape(1, H), W)

    y1 = run_pre(x_c, W1)
    a1 = edge_kernel(y1, src_all, dst_all, zerosD).reshape(2 * A, H)
    y2 = run_mid(a1, b1, W2)
    a2 = edge_kernel(y2, src_all, dst_all, zerosD).reshape(2 * A, H)
    y3 = run_mid(a2, b2, W3)
    a3 = edge_kernel(y3, src_all, dst_all, zerosD).reshape(2 * A, H)

    batch_c = jnp.full((2, A), 1000, jnp.int32)
    batch_c = (batch_c.at[0, :N].set(batch1).at[1, :N].set(batch2 + G)
               .reshape(nblk, 1, _BR))
    pooled = pl.pallas_call(
        _pool_body,
        grid=(nblk,),
        in_specs=[
            pl.BlockSpec((_BR, H), lambda i: (i, 0)),
            pl.BlockSpec((_BR, H), lambda i: (i, 0)),
            pl.BlockSpec((1, H), lambda i: (0, 0)),
            pl.BlockSpec((1, 1, _BR), lambda i: (i, 0, 0)),
        ],
        out_specs=pl.BlockSpec((128, H + 128), lambda i: (0, 0)),
        out_shape=jax.ShapeDtypeStruct((128, H + 128), jnp.float32),
        scratch_shapes=[pltpu.VMEM((128, H + 128), jnp.float32)],
    )(a3, deg16, b3.reshape(1, H), batch_c)

    OUT = fc3_W.shape[1]
    fc3_Wp = jnp.pad(fc3_W, ((0, 0), (0, 128 - OUT)))
    fc3_bp = jnp.pad(fc3_b, (0, 128 - OUT)).reshape(1, 128)
    out = pl.pallas_call(
        _head_body,
        out_shape=jax.ShapeDtypeStruct((G, 128), jnp.float32),
    )(pooled, fc1_W, fc1_b.reshape(1, -1), fc2_W, fc2_b.reshape(1, -1),
      fc3_Wp, fc3_bp)
    return out[:, :OUT]


# speedup vs baseline: 13.1503x; 1.2067x over previous
import functools

import jax
import jax.numpy as jnp
from jax import lax
from jax.experimental import pallas as pl
from jax.experimental.pallas import tpu as pltpu
from jax.experimental.pallas import tpu_sc as plsc

G = 64


_NSUB = 16
_CHUNK = 128


def _sc_mesh():
    return plsc.VectorSubcoreMesh(core_axis_name="c", subcore_axis_name="s")


def _make_deg_kernel(C, A, Dd):
    nz = A // (_NSUB * _CHUNK)

    @functools.partial(
        pl.kernel,
        mesh=_sc_mesh(),
        out_type=jax.ShapeDtypeStruct((2, A, Dd), jnp.float32),
        scratch_types=[
            pltpu.VMEM((_CHUNK,), jnp.int32),
            pltpu.VMEM((_CHUNK,), jnp.int32),
            pltpu.VMEM((_CHUNK, Dd), jnp.float32),
            pltpu.VMEM((_CHUNK, Dd), jnp.float32),
            pltpu.VMEM_SHARED((A, Dd), jnp.float32),
            pltpu.SemaphoreType.DMA,
            pltpu.SemaphoreType.DMA,
        ],
    )
    def deg_kernel(dst_hbm, zeros_hbm, ones_hbm, deg_out,
                   dstA, dstB, rows_v, ones_v, acc, sDA, sDB):
        c = lax.axis_index("c")
        s = lax.axis_index("s")
        pltpu.sync_copy(zeros_hbm, rows_v)
        for k in range(nz):
            pltpu.sync_copy(rows_v, acc.at[pl.ds((s * nz + k) * _CHUNK, _CHUNK)])
        pltpu.sync_copy(ones_hbm, ones_v)
        pltpu.async_copy(dst_hbm.at[c, s, 0], dstA, sDA)
        pltpu.async_copy(dst_hbm.at[c, s, 1], dstB, sDB)
        plsc.subcore_barrier()
        pltpu.make_async_copy(dst_hbm.at[c, s, 0], dstA, sDA).wait()

        def body(t, carry):
            j0 = 2 * t
            pltpu.sync_copy(ones_v, acc.at[dstA], add=True)
            pltpu.async_copy(dst_hbm.at[c, s, j0 + 2], dstA, sDA)
            pltpu.make_async_copy(dst_hbm.at[c, s, 0], dstB, sDB).wait()
            pltpu.sync_copy(ones_v, acc.at[dstB], add=True)
            pltpu.async_copy(dst_hbm.at[c, s, j0 + 3], dstB, sDB)
            pltpu.make_async_copy(dst_hbm.at[c, s, 0], dstA, sDA).wait()
            return carry

        lax.fori_loop(0, C // 2, body, 0)
        pltpu.make_async_copy(dst_hbm.at[c, s, 0], dstB, sDB).wait()
        plsc.subcore_barrier()
        for k in range(nz):
            r0 = (s * nz + k) * _CHUNK
            pltpu.sync_copy(acc.at[pl.ds(r0, _CHUNK)], rows_v)
            pltpu.sync_copy(rows_v, deg_out.at[c, pl.ds(r0, _CHUNK)])

    return deg_kernel


def _make_edge_kernel(C, A, Dd):
    nz = A // (_NSUB * _CHUNK)

    @functools.partial(
        pl.kernel,
        mesh=_sc_mesh(),
        out_type=jax.ShapeDtypeStruct((2, A, Dd), jnp.float32),
        scratch_types=[
            pltpu.VMEM((_CHUNK,), jnp.int32),
            pltpu.VMEM((_CHUNK,), jnp.int32),
            pltpu.VMEM((_CHUNK,), jnp.int32),
            pltpu.VMEM((_CHUNK,), jnp.int32),
            pltpu.VMEM((_CHUNK, Dd), jnp.float32),
            pltpu.VMEM((_CHUNK, Dd), jnp.float32),
            pltpu.VMEM_SHARED((A, Dd), jnp.float32),
            pltpu.SemaphoreType.DMA,
            pltpu.SemaphoreType.DMA,
            pltpu.SemaphoreType.DMA,
            pltpu.SemaphoreType.DMA,
            pltpu.SemaphoreType.DMA,
            pltpu.SemaphoreType.DMA,
        ],
    )
    def edge_kernel(y_hbm, src_hbm, dst_hbm, zeros_hbm, acc_out,
                    srcA, dstA, srcB, dstB, rowsA, rowsB, acc,
                    gsA, gsB, sSA, sDA, sSB, sDB):
        c = lax.axis_index("c")
        s = lax.axis_index("s")
        pltpu.sync_copy(zeros_hbm, rowsA)
        for k in range(nz):
            pltpu.sync_copy(rowsA, acc.at[pl.ds((s * nz + k) * _CHUNK, _CHUNK)])
        pltpu.async_copy(src_hbm.at[c, s, 0], srcA, sSA)
        pltpu.async_copy(dst_hbm.at[c, s, 0], dstA, sDA)
        pltpu.async_copy(src_hbm.at[c, s, 1], srcB, sSB)
        pltpu.async_copy(dst_hbm.at[c, s, 1], dstB, sDB)
        plsc.subcore_barrier()
        pltpu.make_async_copy(src_hbm.at[c, s, 0], srcA, sSA).wait()
        pltpu.make_async_copy(dst_hbm.at[c, s, 0], dstA, sDA).wait()
        pltpu.async_copy(y_hbm.at[srcA], rowsA, gsA)

        def body(t, carry):
            j0 = 2 * t
            pltpu.make_async_copy(src_hbm.at[c, s, 0], srcB, sSB).wait()
            pltpu.make_async_copy(dst_hbm.at[c, s, 0], dstB, sDB).wait()
            pltpu.async_copy(y_hbm.at[srcB], rowsB, gsB)
            pltpu.make_async_copy(y_hbm.at[srcA], rowsA, gsA).wait()
            pltpu.sync_copy(rowsA, acc.at[dstA], add=True)
            pltpu.async_copy(src_hbm.at[c, s, j0 + 2], srcA, sSA)
            pltpu.async_copy(dst_hbm.at[c, s, j0 + 2], dstA, sDA)
            pltpu.make_async_copy(src_hbm.at[c, s, 0], srcA, sSA).wait()
            pltpu.make_async_copy(dst_hbm.at[c, s, 0], dstA, sDA).wait()
            pltpu.async_copy(y_hbm.at[srcA], rowsA, gsA)
            pltpu.make_async_copy(y_hbm.at[srcB], rowsB, gsB).wait()
            pltpu.sync_copy(rowsB, acc.at[dstB], add=True)
            pltpu.async_copy(src_hbm.at[c, s, j0 + 3], srcB, sSB)
            pltpu.async_copy(dst_hbm.at[c, s, j0 + 3], dstB, sDB)
            return carry

        lax.fori_loop(0, C // 2, body, 0)
        pltpu.make_async_copy(y_hbm.at[srcA], rowsA, gsA).wait()
        pltpu.make_async_copy(src_hbm.at[c, s, 0], srcB, sSB).wait()
        pltpu.make_async_copy(dst_hbm.at[c, s, 0], dstB, sDB).wait()
        plsc.subcore_barrier()
        for k in range(nz):
            r0 = (s * nz + k) * _CHUNK
            pltpu.sync_copy(acc.at[pl.ds(r0, _CHUNK)], rowsA)
            pltpu.sync_copy(rowsA, acc_out.at[c, pl.ds(r0, _CHUNK)])

    return edge_kernel


_BR = 1024


def _dinv(d_ref):
    return lax.rsqrt(jnp.maximum(d_ref[:, 0:1], 1.0))


def _mm_pre_body(x_ref, d_ref, w_ref, o_ref):
    o_ref[...] = jnp.dot(x_ref[...] * _dinv(d_ref), w_ref[...],
                         preferred_element_type=jnp.float32)


def _mm_mid_body(a_ref, d_ref, b_ref, w_ref, o_ref):
    dinv = _dinv(d_ref)
    h = jnp.maximum(a_ref[...] * dinv + b_ref[...], 0.0)
    o_ref[...] = jnp.dot(h * dinv, w_ref[...],
                         preferred_element_type=jnp.float32)


def _pool_body(a_ref, d_ref, b_ref, batch_ref, o_ref, acc_ref):
    i = pl.program_id(0)

    @pl.when(i == 0)
    def _():
        acc_ref[...] = jnp.zeros_like(acc_ref)

    h3 = a_ref[...] * _dinv(d_ref) + b_ref[...]
    he = jnp.concatenate(
        [h3, jnp.ones((h3.shape[0], 128), jnp.float32)], axis=1)
    seg = batch_ref[0]
    qt = (lax.broadcasted_iota(jnp.int32, (128, h3.shape[0]), 0)
          == seg).astype(jnp.float32)
    acc_ref[...] += jnp.dot(qt, he, preferred_element_type=jnp.float32)

    @pl.when(i == pl.num_programs(0) - 1)
    def _():
        o_ref[...] = acc_ref[...]


def _head_body(p_ref, w1_ref, b1_ref, w2_ref, b2_ref, w3_ref, b3_ref, o_ref):
    pooled = p_ref[...]
    cnt = jnp.maximum(pooled[:, 128:129], 1.0)
    mean = pooled[:, :128] / cnt
    h = jnp.concatenate([mean[:G], mean[G:2 * G]], axis=1)
    h = jnp.maximum(jnp.dot(h, w1_ref[...],
                            preferred_element_type=jnp.float32) + b1_ref[...],
                    0.0)
    h = jnp.maximum(jnp.dot(h, w2_ref[...],
                            preferred_element_type=jnp.float32) + b2_ref[...],
                    0.0)
    o_ref[...] = jnp.dot(h, w3_ref[...],
                         preferred_element_type=jnp.float32) + b3_ref[...]


def kernel(x1, edge_index1, batch1, x2, edge_index2, batch2,
           W1, b1, W2, b2, W3, b3,
           fc1_W, fc1_b, fc2_W, fc2_b, fc3_W, fc3_b):
    N, D = x1.shape
    H = W1.shape[1]
    E = edge_index1.shape[1]
    Etot = E + N
    C = -(-Etot // (_NSUB * _CHUNK))
    C += C % 2
    EP = _NSUB * C * _CHUNK
    A = (N // (_NSUB * _CHUNK) + 1) * _NSUB * _CHUNK

    loop = jnp.arange(N, dtype=jnp.int32)

    def prep(ei, off):
        src = jnp.concatenate([ei[0] + off, loop + off])
        dst = jnp.concatenate([ei[1], loop])
        src = jnp.pad(src, (0, EP - Etot), constant_values=off)
        dst = jnp.pad(dst, (0, EP - Etot), constant_values=N)
        src = jnp.concatenate(
            [src.reshape(_NSUB, C, _CHUNK),
             jnp.full((_NSUB, 2, _CHUNK), off, jnp.int32)], axis=1)
        dst = jnp.concatenate(
            [dst.reshape(_NSUB, C, _CHUNK),
             jnp.full((_NSUB, 2, _CHUNK), N, jnp.int32)], axis=1)
        return src, dst

    s1, d1 = prep(edge_index1, 0)
    s2, d2 = prep(edge_index2, A)
    src_all = jnp.stack([s1, s2])
    dst_all = jnp.stack([d1, d2])

    zerosD = jnp.zeros((_CHUNK, D), jnp.float32)
    edge_kernel = _make_edge_kernel(C, A, D)

    onesD = jnp.ones((_CHUNK, D), jnp.float32)
    deg16 = _make_deg_kernel(C, A, D)(dst_all, zerosD, onesD).reshape(2 * A, D)

    x_c = jnp.zeros((2, A, D), jnp.float32)
    x_c = x_c.at[0, :N].set(x1).at[1, :N].set(x2).reshape(2 * A, D)
    nblk = (2 * A) // _BR

    def run_pre(x, W):
        return pl.pallas_call(
            _mm_pre_body,
            grid=(nblk,),
            in_specs=[
                pl.BlockSpec((_BR, D), lambda i: (i, 0)),
                pl.BlockSpec((_BR, D), lambda i: (i, 0)),
                pl.BlockSpec((D, H), lambda i: (0, 0)),
            ],
            out_specs=pl.BlockSpec((_BR, H), lambda i: (i, 0)),
            out_shape=jax.ShapeDtypeStruct((2 * A, H), jnp.float32),
        )(x, deg16, W)

    def run_mid(a, b, W):
        return pl.pallas_call(
            _mm_mid_body,
            grid=(nblk,),
            in_specs=[
                pl.BlockSpec((_BR, H), lambda i: (i, 0)),
                pl.BlockSpec((_BR, H), lambda i: (i, 0)),
                pl.BlockSpec((1, H), lambda i: (0, 0)),
                pl.BlockSpec((H, H), lambda i: (0, 0)),
            ],
            out_specs=pl.BlockSpec((_BR, H), lambda i: (i, 0)),
            out_shape=jax.ShapeDtypeStruct((2 * A, H), jnp.float32),
        )(a, deg16, b.reshape(1, H), W)

    y1 = run_pre(x_c, W1)
    a1 = edge_kernel(y1, src_all, dst_all, zerosD).reshape(2 * A, H)
    y2 = run_mid(a1, b1, W2)
    a2 = edge_kernel(y2, src_all, dst_all, zerosD).reshape(2 * A, H)
    y3 = run_mid(a2, b2, W3)
    a3 = edge_kernel(y3, src_all, dst_all, zerosD).reshape(2 * A, H)

    batch_c = jnp.full((2, A), 1000, jnp.int32)
    batch_c = (batch_c.at[0, :N].set(batch1).at[1, :N].set(batch2 + G)
               .reshape(nblk, 1, _BR))
    pooled = pl.pallas_call(
        _pool_body,
        grid=(nblk,),
        in_specs=[
            pl.BlockSpec((_BR, H), lambda i: (i, 0)),
            pl.BlockSpec((_BR, H), lambda i: (i, 0)),
            pl.BlockSpec((1, H), lambda i: (0, 0)),
            pl.BlockSpec((1, 1, _BR), lambda i: (i, 0, 0)),
        ],
        out_specs=pl.BlockSpec((128, H + 128), lambda i: (0, 0)),
        out_shape=jax.ShapeDtypeStruct((128, H + 128), jnp.float32),
        scratch_shapes=[pltpu.VMEM((128, H + 128), jnp.float32)],
    )(a3, deg16, b3.reshape(1, H), batch_c)

    OUT = fc3_W.shape[1]
    fc3_Wp = jnp.pad(fc3_W, ((0, 0), (0, 128 - OUT)))
    fc3_bp = jnp.pad(fc3_b, (0, 128 - OUT)).reshape(1, 128)
    out = pl.pallas_call(
        _head_body,
        out_shape=jax.ShapeDtypeStruct((G, 128), jnp.float32),
    )(pooled, fc1_W, fc1_b.reshape(1, -1), fc2_W, fc2_b.reshape(1, -1),
      fc3_Wp, fc3_bp)
    return out[:, :OUT]
